# baseline (speedup 1.0000x reference)
import functools

import jax
import jax.numpy as jnp
from jax import lax
from jax.experimental import pallas as pl
from jax.experimental.pallas import tpu as pltpu
from jax.experimental.pallas import tpu_sc as plsc

_HALF = 8192
_K = 32
_NE = 1000000
_W128 = 128
_RPE = _W128 // _K
_NROW = _NE // _RPE
_NC = 2
_NS = 16
_L = 16
_NW = _NC * _NS
_BPW = _HALF // _NW
_ROWS = 5 * _BPW
_CH = 128
_NCH = _ROWS // _CH

_SPECS = ((0, 0), (0, 1), (0, 2), (1, 0), (1, 2))


def _body(x_hbm, e_hbm, r_hbm, o_hs, o_ls, o_ts, o_hcs, o_tcs,
          xp_v, xn_v, ev_v, rv_v, gath_v, rows_v, gsem, wsem):
    w = lax.axis_index("s") * _NC + lax.axis_index("c")
    outs = (o_hs, o_ls, o_ts, o_hcs, o_tcs)
    base = w * (_BPW * 3)
    pltpu.sync_copy(x_hbm.at[pl.ds(base, _BPW * 3)], xp_v)
    pltpu.sync_copy(x_hbm.at[pl.ds(_HALF * 3 + base, _BPW * 3)], xn_v)
    lanes = lax.iota(jnp.int32, _L)
    lanes3 = lanes * 3
    halves = (xp_v, xn_v)
    for slot, (half_sel, col) in enumerate(_SPECS):
        for i in range(_BPW // _L):
            v = plsc.load_gather(halves[half_sel], [lanes3 + (i * _L * 3 + col)])
            rv_v[pl.ds(slot * _BPW + i * _L, _L)] = v >> 2
            ev_v[pl.ds(slot * _BPW + i * _L, _L)] = (v & 3) * _K

    lanes32 = lanes * _K

    def extract(c):
        buf = c % 2

        def block(b, carry):
            j0 = b * _L
            rowv = j0 + lanes
            colv = ev_v[pl.ds(c * _CH + j0, _L)]
            for k in range(_K):
                v = plsc.load_gather(gath_v.at[buf], [rowv, colv + k])
                plsc.store_scatter(
                    rows_v, [lanes32 + (c * _CH * _K + j0 * _K + k)], v)
            return carry

        lax.fori_loop(0, _CH // _L, block, 0)

    handles = []
    for c in range(_NCH):
        tab = r_hbm if c // 2 == 1 else e_hbm
        handles.append(pltpu.async_copy(
            tab.at[rv_v.at[pl.ds(c * _CH, _CH)]],
            gath_v.at[c % 2],
            gsem))
        if c >= 1:
            handles[c - 1].wait()
            extract(c - 1)
    handles[_NCH - 1].wait()
    extract(_NCH - 1)

    whandles = []
    for slot in range(5):
        whandles.append(pltpu.async_copy(
            rows_v.at[pl.ds(slot * _BPW * _K, _BPW * _K)],
            outs[slot].at[pl.ds(w * _BPW * _K, _BPW * _K)],
            wsem))
    for h in whandles:
        h.wait()


@jax.jit
def _gather5(x_flat, emb_e2, emb_r2):
    mesh = plsc.VectorSubcoreMesh(core_axis_name="c", subcore_axis_name="s")
    f = pl.kernel(
        _body,
        out_type=[jax.ShapeDtypeStruct((_HALF * _K,), jnp.float32)] * 5,
        mesh=mesh,
        compiler_params=pltpu.CompilerParams(
            needs_layout_passes=False, use_tc_tiling_on_sc=False),
        scratch_types=[
            pltpu.VMEM((_BPW * 3,), jnp.int32),
            pltpu.VMEM((_BPW * 3,), jnp.int32),
            pltpu.VMEM((_ROWS,), jnp.int32),
            pltpu.VMEM((_ROWS,), jnp.int32),
            pltpu.VMEM((2, _CH, _W128), jnp.float32),
            pltpu.VMEM((_ROWS * _K,), jnp.float32),
            pltpu.SemaphoreType.DMA,
            pltpu.SemaphoreType.DMA,
        ],
    )
    return f(x_flat, emb_e2, emb_r2)


def kernel(X, emb_E, emb_R):
    flats = _gather5(
        X.reshape(-1),
        emb_E.reshape(_NROW, _W128),
        emb_R.reshape(_NROW, _W128),
    )
    return tuple(v.reshape(_HALF, _K) for v in flats)


# speedup vs baseline: 1.0538x; 1.0538x over previous
import functools

import jax
import jax.numpy as jnp
from jax import lax
from jax.experimental import pallas as pl
from jax.experimental.pallas import tpu as pltpu
from jax.experimental.pallas import tpu_sc as plsc

_HALF = 8192
_K = 32
_NC = 2
_NS = 16
_L = 16
_NW = _NC * _NS
_BPW = _HALF // _NW
_NCH = 2
_CH = _BPW // _NCH

_SPECS = ((0, 0), (0, 1), (0, 2), (1, 0), (1, 2))
_TABLES = (0, 1, 0, 0, 0)


def _body(x_hbm, emb_e, emb_r, o_hs, o_ls, o_ts, o_hcs, o_tcs,
          xp_v, xn_v, idx_v, rows_v, gsem, wsem):
    w = lax.axis_index("s") * _NC + lax.axis_index("c")
    outs = (o_hs, o_ls, o_ts, o_hcs, o_tcs)
    tables = (emb_e, emb_r)
    base = w * (_BPW * 3)
    pltpu.sync_copy(x_hbm.at[pl.ds(base, _BPW * 3)], xp_v)
    pltpu.sync_copy(x_hbm.at[pl.ds(_HALF * 3 + base, _BPW * 3)], xn_v)
    lanes3 = lax.iota(jnp.int32, _L) * 3
    halves = (xp_v, xn_v)
    vecs_per_chunk = _CH // _L
    for i in range(_BPW // _L):
        off = lanes3 + (i * _L * 3)
        for slot, (half_sel, col) in enumerate(_SPECS):
            v = plsc.load_gather(halves[half_sel], [off + col])
            idx_v[slot, i // vecs_per_chunk,
                  pl.ds((i % vecs_per_chunk) * _L, _L)] = v
    handles = []
    for slot in range(5):
        table = tables[_TABLES[slot]]
        for j in range(_NCH):
            handles.append(pltpu.async_copy(
                table.at[idx_v.at[slot, j]],
                rows_v.at[slot, pl.ds(j * _CH, _CH), :],
                gsem))
    for h in handles:
        h.wait()
    whandles = []
    for slot in range(5):
        whandles.append(pltpu.async_copy(
            rows_v.at[slot], outs[slot].at[pl.ds(w * _BPW, _BPW), :], wsem))
    for h in whandles:
        h.wait()


@jax.jit
def _gather5(x_flat, emb_e, emb_r):
    mesh = plsc.VectorSubcoreMesh(core_axis_name="c", subcore_axis_name="s")
    f = pl.kernel(
        _body,
        out_type=[jax.ShapeDtypeStruct((_HALF, _K), jnp.float32)] * 5,
        mesh=mesh,
        compiler_params=pltpu.CompilerParams(
            needs_layout_passes=False, use_tc_tiling_on_sc=False),
        scratch_types=[
            pltpu.VMEM((_BPW * 3,), jnp.int32),
            pltpu.VMEM((_BPW * 3,), jnp.int32),
            pltpu.VMEM((5, _NCH, _CH), jnp.int32),
            pltpu.VMEM((5, _BPW, _K), jnp.float32),
            pltpu.SemaphoreType.DMA,
            pltpu.SemaphoreType.DMA,
        ],
    )
    return f(x_flat, emb_e, emb_r)


def kernel(X, emb_E, emb_R):
    e_hs, e_ls, e_ts, e_hcs, e_tcs = _gather5(X.reshape(-1), emb_E, emb_R)
    return (e_hs, e_ls, e_ts, e_hcs, e_tcs)
